# baseline (speedup 1.0000x reference)
import functools

import jax
import jax.numpy as jnp
from jax import lax
from jax.experimental import pallas as pl
from jax.experimental.pallas import tpu as pltpu
from jax.experimental.pallas import tpu_sc as plsc

B = 4096
L = 200
WD = 50
WDP = 128
PD = 5
PRW = 10
PRWP = 16
MAXLEN = 200
N = B * L
NC = 2
NS = 16
NW = NC * NS
PER_W = N // NW
CHUNK = 256
K = CHUNK // 128
KA = 3 * K
ITERS = PER_W // CHUNK
ROWS128 = N // 128
W_ROWS = PER_W // 128
NDIS = 2 * MAXLEN
NDEP = 56
DEP2_OFF = NDIS * NDIS
POS2_OFF = DEP2_OFF + NDEP * NDEP
AFF_OFF = ROWS128
BITS_OFF = AFF_OFF + 3 * ROWS128
NCHUNK = N // CHUNK

_mesh = plsc.VectorSubcoreMesh(core_axis_name="c", subcore_axis_name="s")


@functools.partial(
    pl.kernel,
    out_type=[
        jax.ShapeDtypeStruct((N, WDP), jnp.float32),
        jax.ShapeDtypeStruct((3 * N, PRWP), jnp.float32),
        jax.ShapeDtypeStruct((N, WDP), jnp.float32),
        jax.ShapeDtypeStruct((N, WDP), jnp.float32),
    ],
    mesh=_mesh,
    compiler_params=pltpu.CompilerParams(use_tc_tiling_on_sc=False),
    scratch_types=[
        pltpu.VMEM((K, 128), jnp.int32),
        pltpu.VMEM((1, 16), jnp.int32),
        pltpu.VMEM((KA, 128), jnp.int32),
        pltpu.VMEM((16,), jnp.int32),
        pltpu.VMEM((CHUNK, WDP), jnp.float32),
        pltpu.VMEM((CHUNK, WDP), jnp.float32),
        pltpu.VMEM((16, WDP), jnp.float32),
        pltpu.VMEM_SHARED((NS, 16, WDP), jnp.float32),
        pltpu.VMEM((3 * CHUNK, PRWP), jnp.float32),
        pltpu.SemaphoreType.DMA,
    ],
)
def _sc_embed(idx_h, wt_h, comb_h,
              x_h, aff_h, subj_h, obj_h,
              widx, bbuf, aidx, zidx,
              xrows, zrows, z16, zshr, arows, sem):
    cid = lax.axis_index("c")
    sid = lax.axis_index("s")
    wid = sid * NC + cid
    zero = jnp.zeros((16,), jnp.int32)

    zidx[pl.ds(0, 16)] = zero
    pltpu.async_copy(wt_h.at[zidx], z16, sem).wait()
    pltpu.sync_copy(z16, zshr.at[sid])
    for k in range(CHUNK // 16):
        pltpu.sync_copy(zshr.at[sid], zrows.at[pl.ds(16 * k, 16)])

    def step(it, carry):
        base_row = wid * W_ROWS + it * K
        base = base_row * 128
        chunk_id = wid * ITERS + it

        pltpu.sync_copy(idx_h.at[pl.ds(base_row, K)], widx)
        pltpu.sync_copy(
            idx_h.at[pl.ds(BITS_OFF + chunk_id // 8, 1),
                     pl.ds((chunk_id % 8) * 16, 16)], bbuf)
        pltpu.sync_copy(
            idx_h.at[pl.ds(AFF_OFF + 3 * base_row, KA)], aidx)

        cps = []
        for j in range(K):
            cps.append(pltpu.async_copy(
                wt_h.at[widx.at[j]], xrows.at[pl.ds(j * 128, 128)], sem))
        for j in range(KA):
            cps.append(pltpu.async_copy(
                comb_h.at[aidx.at[j]], arows.at[pl.ds(j * 128, 128)], sem))
        for cp in cps:
            cp.wait()

        rows_sl = pl.ds(base, CHUNK)
        pltpu.sync_copy(xrows, x_h.at[rows_sl])
        pltpu.sync_copy(arows, aff_h.at[pl.ds(3 * base, 3 * CHUNK)])
        pltpu.sync_copy(zrows, subj_h.at[rows_sl])
        pltpu.sync_copy(zrows, obj_h.at[rows_sl])

        bv = bbuf[0, pl.ds(0, 16)]
        for g in range(CHUNK // 16):
            bg = bv[g]

            @pl.when(bg != 0)
            def _patch(bg=bg, g=g):
                def lane_body(lane, carry):
                    p = g * 16 + lane

                    @pl.when((bg >> lane) & 1 != 0)
                    def _subj():
                        pltpu.sync_copy(xrows.at[pl.ds(p, 1)],
                                        subj_h.at[pl.ds(base + p, 1)])

                    @pl.when((bg >> (16 + lane)) & 1 != 0)
                    def _obj():
                        pltpu.sync_copy(xrows.at[pl.ds(p, 1)],
                                        obj_h.at[pl.ds(base + p, 1)])
                    return carry
                lax.fori_loop(0, 16, lane_body, 0)
        return carry

    lax.fori_loop(0, ITERS, step, 0)


def kernel(word, pos1, pos2, subj_deprel, obj_deprel, subj_dis, obj_dis,
           word_table, pos_table, dis_table, dep_table):
    wt128 = jnp.pad(word_table, ((0, 0), (0, WDP - WD)))
    def pairs(t, v):
        a = jnp.broadcast_to(t[:, None, :], (v, v, PD))
        b = jnp.broadcast_to(t[None, :, :], (v, v, PD))
        return jnp.concatenate([a, b], axis=-1).reshape(v * v, PRW)
    comb16 = jnp.pad(
        jnp.concatenate([pairs(dis_table, NDIS), pairs(dep_table, NDEP),
                         pairs(pos_table, NDIS)], axis=0),
        ((0, 0), (0, PRWP - PRW)))
    aff_idx = jnp.stack([
        subj_dis * NDIS + obj_dis,
        DEP2_OFF + subj_deprel * NDEP + obj_deprel,
        POS2_OFF + pos1 * NDIS + pos2,
    ], axis=-1).reshape(3 * ROWS128, 128)
    lanebit = (1 << jnp.arange(16, dtype=jnp.int32))
    pack = lambda m: jnp.sum(
        jnp.where(m.reshape(-1, 16), lanebit, 0), axis=1).astype(jnp.int32)
    bits = (pack(pos1 == MAXLEN) |
            (pack(pos2 == MAXLEN) << 16)).reshape(-1, 128)
    idx_all = jnp.concatenate(
        [word.reshape(ROWS128, 128), aff_idx, bits], axis=0)
    x, aff, subj, obj = _sc_embed(idx_all, wt128, comb16)
    zero_bait = word_table[0, 0] * 0.0
    return (x[:, :WD].reshape(B, L, WD),
            aff[:, :PRW].reshape(B, L, 6 * PD) + zero_bait,
            subj[:, :WD].reshape(B, L, WD),
            obj[:, :WD].reshape(B, L, WD))


# speedup vs baseline: 1.1462x; 1.1462x over previous
import functools

import jax
import jax.numpy as jnp
from jax import lax
from jax.experimental import pallas as pl
from jax.experimental.pallas import tpu as pltpu
from jax.experimental.pallas import tpu_sc as plsc

B = 4096
L = 200
WD = 50
WDP = 128
PD = 5
PRW = 10
PRWP = 16
MAXLEN = 200
N = B * L
NC = 2
NS = 16
NW = NC * NS
PER_W = N // NW
CHUNK = 512
K = CHUNK // 128
KA = 3 * K
ITERS = PER_W // CHUNK
ROWS128 = N // 128
W_ROWS = PER_W // 128
NDIS = 2 * MAXLEN
NDEP = 56
DEP2_OFF = NDIS * NDIS
POS2_OFF = DEP2_OFF + NDEP * NDEP
AFF_OFF = ROWS128
BITS_OFF = AFF_OFF + 3 * ROWS128
NCHUNK = N // CHUNK

_mesh = plsc.VectorSubcoreMesh(core_axis_name="c", subcore_axis_name="s")


@functools.partial(
    pl.kernel,
    out_type=[
        jax.ShapeDtypeStruct((N, WDP), jnp.float32),
        jax.ShapeDtypeStruct((3 * N, PRWP), jnp.float32),
        jax.ShapeDtypeStruct((N, WDP), jnp.float32),
        jax.ShapeDtypeStruct((N, WDP), jnp.float32),
    ],
    mesh=_mesh,
    compiler_params=pltpu.CompilerParams(use_tc_tiling_on_sc=False),
    scratch_types=[
        pltpu.VMEM((K, 128), jnp.int32),
        pltpu.VMEM((1, 32), jnp.int32),
        pltpu.VMEM((KA, 128), jnp.int32),
        pltpu.VMEM((16,), jnp.int32),
        pltpu.VMEM((CHUNK, WDP), jnp.float32),
        pltpu.VMEM((CHUNK // 2, WDP), jnp.float32),
        pltpu.VMEM((16, WDP), jnp.float32),
        pltpu.VMEM_SHARED((NS, 16, WDP), jnp.float32),
        pltpu.VMEM((3 * CHUNK, PRWP), jnp.float32),
        pltpu.SemaphoreType.DMA,
    ],
)
def _sc_embed(idx_h, wt_h, comb_h,
              x_h, aff_h, subj_h, obj_h,
              widx, bbuf, aidx, zidx,
              xrows, zrows, z16, zshr, arows, sem):
    cid = lax.axis_index("c")
    sid = lax.axis_index("s")
    wid = sid * NC + cid
    zero = jnp.zeros((16,), jnp.int32)

    zidx[pl.ds(0, 16)] = zero
    pltpu.async_copy(wt_h.at[zidx], z16, sem).wait()
    pltpu.sync_copy(z16, zshr.at[sid])
    for k in range(CHUNK // 32):
        pltpu.sync_copy(zshr.at[sid], zrows.at[pl.ds(16 * k, 16)])

    def step(it, carry):
        base_row = wid * W_ROWS + it * K
        base = base_row * 128
        chunk_id = wid * ITERS + it

        pltpu.sync_copy(idx_h.at[pl.ds(base_row, K)], widx)
        pltpu.sync_copy(
            idx_h.at[pl.ds(BITS_OFF + chunk_id // 4, 1),
                     pl.ds((chunk_id % 4) * 32, 32)], bbuf)
        pltpu.sync_copy(
            idx_h.at[pl.ds(AFF_OFF + 3 * base_row, KA)], aidx)

        cps = []
        for j in range(K):
            cps.append(pltpu.async_copy(
                wt_h.at[widx.at[j]], xrows.at[pl.ds(j * 128, 128)], sem))
        for j in range(KA):
            cps.append(pltpu.async_copy(
                comb_h.at[aidx.at[j]], arows.at[pl.ds(j * 128, 128)], sem))
        for cp in cps:
            cp.wait()

        rows_sl = pl.ds(base, CHUNK)
        pltpu.sync_copy(xrows, x_h.at[rows_sl])
        pltpu.sync_copy(arows, aff_h.at[pl.ds(3 * base, 3 * CHUNK)])
        half = CHUNK // 2
        pltpu.sync_copy(zrows, subj_h.at[pl.ds(base, half)])
        pltpu.sync_copy(zrows, subj_h.at[pl.ds(base + half, half)])
        pltpu.sync_copy(zrows, obj_h.at[pl.ds(base, half)])
        pltpu.sync_copy(zrows, obj_h.at[pl.ds(base + half, half)])

        bv0 = bbuf[0, pl.ds(0, 16)]
        bv1 = bbuf[0, pl.ds(16, 16)]
        for g in range(CHUNK // 16):
            bg = bv0[g] if g < 16 else bv1[g - 16]

            @pl.when(bg != 0)
            def _patch(bg=bg, g=g):
                def lane_body(lane, carry):
                    p = g * 16 + lane

                    @pl.when((bg >> lane) & 1 != 0)
                    def _subj():
                        pltpu.sync_copy(xrows.at[pl.ds(p, 1)],
                                        subj_h.at[pl.ds(base + p, 1)])

                    @pl.when((bg >> (16 + lane)) & 1 != 0)
                    def _obj():
                        pltpu.sync_copy(xrows.at[pl.ds(p, 1)],
                                        obj_h.at[pl.ds(base + p, 1)])
                    return carry
                lax.fori_loop(0, 16, lane_body, 0)
        return carry

    lax.fori_loop(0, ITERS, step, 0)


def kernel(word, pos1, pos2, subj_deprel, obj_deprel, subj_dis, obj_dis,
           word_table, pos_table, dis_table, dep_table):
    wt128 = jnp.pad(word_table, ((0, 0), (0, WDP - WD)))
    def pairs(t, v):
        a = jnp.broadcast_to(t[:, None, :], (v, v, PD))
        b = jnp.broadcast_to(t[None, :, :], (v, v, PD))
        return jnp.concatenate([a, b], axis=-1).reshape(v * v, PRW)
    comb16 = jnp.pad(
        jnp.concatenate([pairs(dis_table, NDIS), pairs(dep_table, NDEP),
                         pairs(pos_table, NDIS)], axis=0),
        ((0, 0), (0, PRWP - PRW)))
    aff_idx = jnp.stack([
        subj_dis * NDIS + obj_dis,
        DEP2_OFF + subj_deprel * NDEP + obj_deprel,
        POS2_OFF + pos1 * NDIS + pos2,
    ], axis=-1).reshape(3 * ROWS128, 128)
    lanebit = (1 << jnp.arange(16, dtype=jnp.int32))
    pack = lambda m: jnp.sum(
        jnp.where(m.reshape(-1, 16), lanebit, 0), axis=1).astype(jnp.int32)
    bits = (pack(pos1 == MAXLEN) |
            (pack(pos2 == MAXLEN) << 16)).reshape(-1, 128)
    idx_all = jnp.concatenate(
        [word.reshape(ROWS128, 128), aff_idx, bits], axis=0)
    x, aff, subj, obj = _sc_embed(idx_all, wt128, comb16)
    return (x[:, :WD].reshape(B, L, WD),
            aff[:, :PRW].reshape(B, L, 6 * PD),
            subj[:, :WD].reshape(B, L, WD),
            obj[:, :WD].reshape(B, L, WD))


# speedup vs baseline: 1.1631x; 1.0147x over previous
import functools

import jax
import jax.numpy as jnp
from jax import lax
from jax.experimental import pallas as pl
from jax.experimental.pallas import tpu as pltpu
from jax.experimental.pallas import tpu_sc as plsc

B = 4096
L = 200
WD = 50
WDP = 128
WTP = 56
PD = 5
PRW = 10
PRWP = 16
MAXLEN = 200
N = B * L
NC = 2
NS = 16
NW = NC * NS
PER_W = N // NW
CHUNK = 512
K = CHUNK // 128
KA = 3 * K
ITERS = PER_W // CHUNK
ROWS128 = N // 128
W_ROWS = PER_W // 128
NDIS = 2 * MAXLEN
NDEP = 56
DEP2_OFF = NDIS * NDIS
POS2_OFF = DEP2_OFF + NDEP * NDEP
AFF_OFF = ROWS128
BITS_OFF = AFF_OFF + 3 * ROWS128
NCHUNK = N // CHUNK

_mesh = plsc.VectorSubcoreMesh(core_axis_name="c", subcore_axis_name="s")


@functools.partial(
    pl.kernel,
    out_type=[
        jax.ShapeDtypeStruct((N, WDP), jnp.float32),
        jax.ShapeDtypeStruct((3 * N, PRWP), jnp.float32),
        jax.ShapeDtypeStruct((N, WDP), jnp.float32),
        jax.ShapeDtypeStruct((N, WDP), jnp.float32),
    ],
    mesh=_mesh,
    compiler_params=pltpu.CompilerParams(use_tc_tiling_on_sc=False),
    scratch_types=[
        pltpu.VMEM((K, 128), jnp.int32),
        pltpu.VMEM((1, 32), jnp.int32),
        pltpu.VMEM((KA, 128), jnp.int32),
        pltpu.VMEM((16,), jnp.int32),
        pltpu.VMEM((CHUNK, WTP), jnp.float32),
        pltpu.VMEM((CHUNK // 2, WDP), jnp.float32),
        pltpu.VMEM((16, WTP), jnp.float32),
        pltpu.VMEM_SHARED((NS, 16, WTP), jnp.float32),
        pltpu.VMEM((3 * CHUNK, PRWP), jnp.float32),
        pltpu.SemaphoreType.DMA,
    ],
)
def _sc_embed(idx_h, wt_h, comb_h,
              x_h, aff_h, subj_h, obj_h,
              widx, bbuf, aidx, zidx,
              xrows, zrows, z16, zshr, arows, sem):
    cid = lax.axis_index("c")
    sid = lax.axis_index("s")
    wid = sid * NC + cid
    zero = jnp.zeros((16,), jnp.int32)

    zidx[pl.ds(0, 16)] = zero
    pltpu.async_copy(wt_h.at[zidx], z16, sem).wait()
    pltpu.sync_copy(z16, zshr.at[sid])
    for k in range(CHUNK // 32):
        pltpu.sync_copy(zshr.at[sid],
                        zrows.at[pl.ds(16 * k, 16), pl.ds(0, WTP)])

    def step(it, carry):
        base_row = wid * W_ROWS + it * K
        base = base_row * 128
        chunk_id = wid * ITERS + it

        pltpu.sync_copy(idx_h.at[pl.ds(base_row, K)], widx)
        pltpu.sync_copy(
            idx_h.at[pl.ds(BITS_OFF + chunk_id // 4, 1),
                     pl.ds((chunk_id % 4) * 32, 32)], bbuf)
        pltpu.sync_copy(
            idx_h.at[pl.ds(AFF_OFF + 3 * base_row, KA)], aidx)

        cps = []
        for j in range(K):
            cps.append(pltpu.async_copy(
                wt_h.at[widx.at[j]], xrows.at[pl.ds(j * 128, 128)], sem))
        for j in range(KA):
            cps.append(pltpu.async_copy(
                comb_h.at[aidx.at[j]], arows.at[pl.ds(j * 128, 128)], sem))
        for cp in cps:
            cp.wait()

        rows_sl = pl.ds(base, CHUNK)
        pltpu.sync_copy(xrows, x_h.at[rows_sl, pl.ds(0, WTP)])
        pltpu.sync_copy(arows, aff_h.at[pl.ds(3 * base, 3 * CHUNK)])
        half = CHUNK // 2
        pltpu.sync_copy(zrows, subj_h.at[pl.ds(base, half)])
        pltpu.sync_copy(zrows, subj_h.at[pl.ds(base + half, half)])
        pltpu.sync_copy(zrows, obj_h.at[pl.ds(base, half)])
        pltpu.sync_copy(zrows, obj_h.at[pl.ds(base + half, half)])

        bv0 = bbuf[0, pl.ds(0, 16)]
        bv1 = bbuf[0, pl.ds(16, 16)]
        for g in range(CHUNK // 16):
            bg = bv0[g] if g < 16 else bv1[g - 16]

            @pl.when(bg != 0)
            def _patch(bg=bg, g=g):
                def lane_body(lane, carry):
                    p = g * 16 + lane

                    @pl.when((bg >> lane) & 1 != 0)
                    def _subj():
                        pltpu.sync_copy(
                            xrows.at[pl.ds(p, 1)],
                            subj_h.at[pl.ds(base + p, 1), pl.ds(0, WTP)])

                    @pl.when((bg >> (16 + lane)) & 1 != 0)
                    def _obj():
                        pltpu.sync_copy(
                            xrows.at[pl.ds(p, 1)],
                            obj_h.at[pl.ds(base + p, 1), pl.ds(0, WTP)])
                    return carry
                lax.fori_loop(0, 16, lane_body, 0)
        return carry

    lax.fori_loop(0, ITERS, step, 0)


def kernel(word, pos1, pos2, subj_deprel, obj_deprel, subj_dis, obj_dis,
           word_table, pos_table, dis_table, dep_table):
    wt56 = jnp.pad(word_table, ((0, 0), (0, WTP - WD)))
    def pairs(t, v):
        a = jnp.broadcast_to(t[:, None, :], (v, v, PD))
        b = jnp.broadcast_to(t[None, :, :], (v, v, PD))
        return jnp.concatenate([a, b], axis=-1).reshape(v * v, PRW)
    comb16 = jnp.pad(
        jnp.concatenate([pairs(dis_table, NDIS), pairs(dep_table, NDEP),
                         pairs(pos_table, NDIS)], axis=0),
        ((0, 0), (0, PRWP - PRW)))
    aff_idx = jnp.stack([
        subj_dis * NDIS + obj_dis,
        DEP2_OFF + subj_deprel * NDEP + obj_deprel,
        POS2_OFF + pos1 * NDIS + pos2,
    ], axis=-1).reshape(3 * ROWS128, 128)
    lanebit = (1 << jnp.arange(16, dtype=jnp.int32))
    pack = lambda m: jnp.sum(
        jnp.where(m.reshape(-1, 16), lanebit, 0), axis=1).astype(jnp.int32)
    bits = (pack(pos1 == MAXLEN) |
            (pack(pos2 == MAXLEN) << 16)).reshape(-1, 128)
    idx_all = jnp.concatenate(
        [word.reshape(ROWS128, 128), aff_idx, bits], axis=0)
    x, aff, subj, obj = _sc_embed(idx_all, wt56, comb16)
    return (x[:, :WD].reshape(B, L, WD),
            aff[:, :PRW].reshape(B, L, 6 * PD),
            subj[:, :WD].reshape(B, L, WD),
            obj[:, :WD].reshape(B, L, WD))


# speedup vs baseline: 1.1897x; 1.0228x over previous
import functools

import jax
import jax.numpy as jnp
from jax import lax
from jax.experimental import pallas as pl
from jax.experimental.pallas import tpu as pltpu
from jax.experimental.pallas import tpu_sc as plsc

B = 4096
L = 200
WD = 50
WDP = 128
WTP = 56
PD = 5
PRW = 10
PRWP = 16
MAXLEN = 200
N = B * L
NC = 2
NS = 16
NW = NC * NS
PER_W = N // NW
CHUNK = 512
K = CHUNK // 128
KA = 3 * K
ITERS = PER_W // CHUNK
ROWS128 = N // 128
W_ROWS = PER_W // 128
NDIS = 2 * MAXLEN
NDEP = 56
DEP2_OFF = NDIS * NDIS
POS2_OFF = DEP2_OFF + NDEP * NDEP
AFF_OFF = ROWS128
BITS_OFF = AFF_OFF + 3 * ROWS128
NCHUNK = N // CHUNK

_mesh = plsc.VectorSubcoreMesh(core_axis_name="c", subcore_axis_name="s")


@functools.partial(
    pl.kernel,
    out_type=[
        jax.ShapeDtypeStruct((N, WDP), jnp.float32),
        jax.ShapeDtypeStruct((3 * N, PRWP), jnp.float32),
        jax.ShapeDtypeStruct((N, WDP), jnp.float32),
        jax.ShapeDtypeStruct((N, WDP), jnp.float32),
    ],
    mesh=_mesh,
    compiler_params=pltpu.CompilerParams(use_tc_tiling_on_sc=False),
    scratch_types=[
        pltpu.VMEM((K, 128), jnp.int32),
        pltpu.VMEM((1, 32), jnp.int32),
        pltpu.VMEM((KA, 128), jnp.int32),
        pltpu.VMEM((16,), jnp.int32),
        pltpu.VMEM((CHUNK, WTP), jnp.float32),
        pltpu.VMEM((CHUNK // 2, WDP), jnp.float32),
        pltpu.VMEM((16, WTP), jnp.float32),
        pltpu.VMEM_SHARED((NS, 16, WTP), jnp.float32),
        pltpu.VMEM((3 * CHUNK, PRWP), jnp.float32),
        pltpu.SemaphoreType.DMA,
        pltpu.SemaphoreType.DMA,
    ],
)
def _sc_embed(idx_h, wt_h, comb_h,
              x_h, aff_h, subj_h, obj_h,
              widx, bbuf, aidx, zidx,
              xrows, zrows, z16, zshr, arows, sem, zsem):
    cid = lax.axis_index("c")
    sid = lax.axis_index("s")
    wid = sid * NC + cid
    zero = jnp.zeros((16,), jnp.int32)

    zidx[pl.ds(0, 16)] = zero
    pltpu.async_copy(wt_h.at[zidx], z16, sem).wait()
    pltpu.sync_copy(z16, zshr.at[sid])
    for k in range(CHUNK // 32):
        pltpu.sync_copy(zshr.at[sid],
                        zrows.at[pl.ds(16 * k, 16), pl.ds(0, WTP)])

    def step(it, carry):
        base_row = wid * W_ROWS + it * K
        base = base_row * 128
        chunk_id = wid * ITERS + it

        half = CHUNK // 2
        zcps = [
            pltpu.async_copy(zrows, subj_h.at[pl.ds(base, half)], zsem),
            pltpu.async_copy(zrows, subj_h.at[pl.ds(base + half, half)], zsem),
            pltpu.async_copy(zrows, obj_h.at[pl.ds(base, half)], zsem),
            pltpu.async_copy(zrows, obj_h.at[pl.ds(base + half, half)], zsem),
        ]

        pltpu.sync_copy(idx_h.at[pl.ds(base_row, K)], widx)
        pltpu.sync_copy(
            idx_h.at[pl.ds(BITS_OFF + chunk_id // 4, 1),
                     pl.ds((chunk_id % 4) * 32, 32)], bbuf)
        pltpu.sync_copy(
            idx_h.at[pl.ds(AFF_OFF + 3 * base_row, KA)], aidx)

        cps = []
        for j in range(K):
            cps.append(pltpu.async_copy(
                wt_h.at[widx.at[j]], xrows.at[pl.ds(j * 128, 128)], sem))
        for j in range(KA):
            cps.append(pltpu.async_copy(
                comb_h.at[aidx.at[j]], arows.at[pl.ds(j * 128, 128)], sem))
        for cp in cps:
            cp.wait()

        rows_sl = pl.ds(base, CHUNK)
        pltpu.sync_copy(xrows, x_h.at[rows_sl, pl.ds(0, WTP)])
        pltpu.sync_copy(arows, aff_h.at[pl.ds(3 * base, 3 * CHUNK)])
        for cp in zcps:
            cp.wait()

        bv0 = bbuf[0, pl.ds(0, 16)]
        bv1 = bbuf[0, pl.ds(16, 16)]
        for g in range(CHUNK // 16):
            bg = bv0[g] if g < 16 else bv1[g - 16]

            @pl.when(bg != 0)
            def _patch(bg=bg, g=g):
                def lane_body(lane, carry):
                    p = g * 16 + lane

                    @pl.when((bg >> lane) & 1 != 0)
                    def _subj():
                        pltpu.sync_copy(
                            xrows.at[pl.ds(p, 1)],
                            subj_h.at[pl.ds(base + p, 1), pl.ds(0, WTP)])

                    @pl.when((bg >> (16 + lane)) & 1 != 0)
                    def _obj():
                        pltpu.sync_copy(
                            xrows.at[pl.ds(p, 1)],
                            obj_h.at[pl.ds(base + p, 1), pl.ds(0, WTP)])
                    return carry
                lax.fori_loop(0, 16, lane_body, 0)
        return carry

    lax.fori_loop(0, ITERS, step, 0)


def kernel(word, pos1, pos2, subj_deprel, obj_deprel, subj_dis, obj_dis,
           word_table, pos_table, dis_table, dep_table):
    wt56 = jnp.pad(word_table, ((0, 0), (0, WTP - WD)))
    def pairs(t, v):
        a = jnp.broadcast_to(t[:, None, :], (v, v, PD))
        b = jnp.broadcast_to(t[None, :, :], (v, v, PD))
        return jnp.concatenate([a, b], axis=-1).reshape(v * v, PRW)
    comb16 = jnp.pad(
        jnp.concatenate([pairs(dis_table, NDIS), pairs(dep_table, NDEP),
                         pairs(pos_table, NDIS)], axis=0),
        ((0, 0), (0, PRWP - PRW)))
    aff_idx = jnp.stack([
        subj_dis * NDIS + obj_dis,
        DEP2_OFF + subj_deprel * NDEP + obj_deprel,
        POS2_OFF + pos1 * NDIS + pos2,
    ], axis=-1).reshape(3 * ROWS128, 128)
    lanebit = (1 << jnp.arange(16, dtype=jnp.int32))
    pack = lambda m: jnp.sum(
        jnp.where(m.reshape(-1, 16), lanebit, 0), axis=1).astype(jnp.int32)
    bits = (pack(pos1 == MAXLEN) |
            (pack(pos2 == MAXLEN) << 16)).reshape(-1, 128)
    idx_all = jnp.concatenate(
        [word.reshape(ROWS128, 128), aff_idx, bits], axis=0)
    x, aff, subj, obj = _sc_embed(idx_all, wt56, comb16)
    return (x[:, :WD].reshape(B, L, WD),
            aff[:, :PRW].reshape(B, L, 6 * PD),
            subj[:, :WD].reshape(B, L, WD),
            obj[:, :WD].reshape(B, L, WD))
